# baseline (speedup 1.0000x reference)
import jax
import jax.numpy as jnp
from jax.experimental import pallas as pl
from jax.experimental.pallas import tpu as pltpu

B, T, D, NT, H, DH, FF, E, K = 64, 576, 768, 4, 12, 64, 3072, 8, 2
HQ = H * NT
BB = 4
FFB = 1536


def _bdot(a, b, dims):
    return jax.lax.dot_general(a, b, (dims, ((), ())),
                               preferred_element_type=jnp.float32)


def _bf(a):
    return a.astype(jnp.bfloat16)


def _split(a):
    hi = _bf(a)
    lo = _bf(a - hi.astype(jnp.float32))
    return hi, lo


def _dot3(a, b, dims):
    ah, al = _split(a)
    bh, bl = _split(b)
    return _bdot(ah, bh, dims) + (_bdot(ah, bl, dims) + _bdot(al, bh, dims))


def _attn_kernel(hs_ref, probe_ref, wq_ref, bq_ref, wk_ref, wv_ref, bv_ref,
                 o_ref, qk_s):
    @pl.when(pl.program_id(0) == 0)
    def _prep():
        q = _bdot(_bf(probe_ref[:]), _bf(wq_ref[:]), ((1,), (1,)))
        q = q + bq_ref[0]
        q_tiled = jnp.tile(q, (H, 1))
        row_h = jax.lax.broadcasted_iota(jnp.int32, (HQ, D), 0) // NT
        col_h = jax.lax.broadcasted_iota(jnp.int32, (HQ, D), 1) // DH
        q_block = jnp.where(row_h == col_h, q_tiled, 0.0)
        qb_h, qb_l = _split(q_block)
        wk_bf = _bf(wk_ref[:])
        qk_s[:] = (_bdot(qb_h, wk_bf, ((1,), (0,)))
                   + _bdot(qb_l, wk_bf, ((1,), (0,))))

    hs_bf = _bf(hs_ref[:])
    hs2 = hs_bf.reshape(BB * T, D)
    qk_h, qk_l = _split(qk_s[:])
    p = _bdot(hs2, jnp.concatenate([qk_h, qk_l], axis=0), ((1,), (1,)))
    s = (p[:, :HQ] + p[:, HQ:]) * 0.125
    s3 = s.reshape(BB, T, HQ)
    m = jnp.max(s3, axis=1, keepdims=True)
    pe = jnp.exp(s3 - m)
    att = pe / jnp.sum(pe, axis=1, keepdims=True)
    att_h, att_l = _split(att)
    v = _bdot(hs2, _bf(wv_ref[:]), ((1,), (1,))) + bv_ref[0]
    v_h, v_l = _split(v.reshape(BB, T, D))
    col_h = jax.lax.broadcasted_iota(jnp.int32, (H, 1, D), 2) // DH
    row_h = jax.lax.broadcasted_iota(jnp.int32, (H, 1, D), 0)
    mask = (col_h == row_h).astype(jnp.float32)
    for j in range(BB):
        o_full = (_bdot(att_h[j], v_h[j], ((0,), (0,)))
                  + (_bdot(att_h[j], v_l[j], ((0,), (0,)))
                     + _bdot(att_l[j], v_h[j], ((0,), (0,)))))
        o_j = jnp.sum(o_full.reshape(H, NT, D) * mask, axis=0)
        o_ref[j * NT:(j + 1) * NT, :] = o_j


def _head_kernel(o_ref, wo_ref, bo_ref, lng_ref, lnb_ref,
                 rw_ref, rb_ref, fc2b_ref,
                 x_ref, w_ref, base_ref, stats_ref, loss_ref):
    attn_out = _dot3(o_ref[:], wo_ref[:], ((1,), (1,)))
    attn_out = attn_out + bo_ref[:]
    mu = jnp.mean(attn_out, axis=1, keepdims=True)
    xc = attn_out - mu
    var = jnp.mean(xc * xc, axis=1, keepdims=True)
    xl = xc / jnp.sqrt(var + 1e-6) * lng_ref[:] + lnb_ref[:]
    x_ref[:] = xl
    logits = _dot3(xl, rw_ref[:], ((1,), (1,))) + rb_ref[:]
    lm = jnp.max(logits, axis=1, keepdims=True)
    pe = jnp.exp(logits - lm)
    probs = pe / jnp.sum(pe, axis=1, keepdims=True)
    cols = jax.lax.broadcasted_iota(jnp.int32, (B * NT, E), 1)
    m1 = jnp.max(probs, axis=1, keepdims=True)
    i1 = jnp.min(jnp.where(probs == m1, cols, E), axis=1, keepdims=True)
    masked = jnp.where(cols == i1, -1.0, probs)
    m2 = jnp.max(masked, axis=1, keepdims=True)
    i2 = jnp.min(jnp.where(masked == m2, cols, E), axis=1, keepdims=True)
    ssum = m1 + m2
    sel1 = cols == i1
    sel2 = cols == i2
    weights = (jnp.where(sel1, m1 / ssum, 0.0)
               + jnp.where(sel2, m2 / ssum, 0.0))
    w_ref[:] = weights
    disp = sel1.astype(jnp.float32) + sel2.astype(jnp.float32)
    stats = jnp.sum(disp, axis=0, keepdims=True)
    stats_ref[:] = stats
    pmean = jnp.sum(probs, axis=0, keepdims=True) * (1.0 / (B * NT))
    loss_ref[:] = (E / (B * NT)) * jnp.sum(stats * pmean,
                                           axis=1, keepdims=True)
    base_ref[:] = attn_out + _bdot(weights, fc2b_ref[:], ((1,), (0,)))


def _ffn_kernel(x_ref, w_ref, base_ref, fc1w_ref, fc1b_ref, fc2w_ref,
                out_ref):
    e = pl.program_id(0)
    fb = pl.program_id(1)

    @pl.when(jnp.logical_and(e == 0, fb == 0))
    def _init():
        out_ref[:] = base_ref[:]

    hpre = _bdot(_bf(x_ref[:]), _bf(fc1w_ref[0]),
                 ((1,), (1,))) + fc1b_ref[0]
    hact = jax.nn.gelu(hpre, approximate=True)
    part = _bdot(_bf(hact), _bf(fc2w_ref[0]), ((1,), (1,)))
    cols = jax.lax.broadcasted_iota(jnp.int32, (B * NT, E), 1)
    w_col = jnp.sum(jnp.where(cols == e, w_ref[:], 0.0), axis=1,
                    keepdims=True)
    out_ref[:] = out_ref[:] + part * w_col


def kernel(hidden_state, probe, in_proj_w, in_proj_b, out_proj_w, out_proj_b,
           ln_g, ln_b, router_w, router_b, fc1_w, fc1_b, fc2_w, fc2_b):
    f32 = jnp.float32
    n = B * NT
    in_proj_b3 = in_proj_b.reshape(3, 1, D)

    o = pl.pallas_call(
        _attn_kernel,
        grid=(B // BB,),
        in_specs=[
            pl.BlockSpec((BB, T, D), lambda i: (i, 0, 0)),
            pl.BlockSpec((NT, D), lambda i: (0, 0)),
            pl.BlockSpec((D, D), lambda i: (0, 0)),
            pl.BlockSpec((1, 1, D), lambda i: (0, 0, 0)),
            pl.BlockSpec((D, D), lambda i: (1, 0)),
            pl.BlockSpec((D, D), lambda i: (2, 0)),
            pl.BlockSpec((1, 1, D), lambda i: (2, 0, 0)),
        ],
        out_specs=pl.BlockSpec((BB * NT, D), lambda i: (i, 0)),
        out_shape=jax.ShapeDtypeStruct((n, D), f32),
        scratch_shapes=[pltpu.VMEM((HQ, D), f32)],
    )(hidden_state, probe.reshape(NT, D), in_proj_w, in_proj_b3,
      in_proj_w, in_proj_w, in_proj_b3)

    x, weights, base, stats, loss = pl.pallas_call(
        _head_kernel,
        grid=(1,),
        in_specs=[
            pl.BlockSpec((n, D), lambda i: (0, 0)),
            pl.BlockSpec((D, D), lambda i: (0, 0)),
            pl.BlockSpec((1, D), lambda i: (0, 0)),
            pl.BlockSpec((1, D), lambda i: (0, 0)),
            pl.BlockSpec((1, D), lambda i: (0, 0)),
            pl.BlockSpec((E, D), lambda i: (0, 0)),
            pl.BlockSpec((1, E), lambda i: (0, 0)),
            pl.BlockSpec((E, D), lambda i: (0, 0)),
        ],
        out_specs=(
            pl.BlockSpec((n, D), lambda i: (0, 0)),
            pl.BlockSpec((n, E), lambda i: (0, 0)),
            pl.BlockSpec((n, D), lambda i: (0, 0)),
            pl.BlockSpec((1, E), lambda i: (0, 0)),
            pl.BlockSpec((1, 1), lambda i: (0, 0)),
        ),
        out_shape=(
            jax.ShapeDtypeStruct((n, D), f32),
            jax.ShapeDtypeStruct((n, E), f32),
            jax.ShapeDtypeStruct((n, D), f32),
            jax.ShapeDtypeStruct((1, E), f32),
            jax.ShapeDtypeStruct((1, 1), f32),
        ),
    )(o, out_proj_w, out_proj_b.reshape(1, D),
      ln_g.reshape(1, D), ln_b.reshape(1, D),
      router_w, router_b.reshape(1, E), fc2_b)

    out = pl.pallas_call(
        _ffn_kernel,
        grid=(E, FF // FFB),
        in_specs=[
            pl.BlockSpec((n, D), lambda e, fb: (0, 0)),
            pl.BlockSpec((n, E), lambda e, fb: (0, 0)),
            pl.BlockSpec((n, D), lambda e, fb: (0, 0)),
            pl.BlockSpec((1, FFB, D), lambda e, fb: (e, fb, 0)),
            pl.BlockSpec((1, 1, FFB), lambda e, fb: (e, 0, fb)),
            pl.BlockSpec((1, D, FFB), lambda e, fb: (e, 0, fb)),
        ],
        out_specs=pl.BlockSpec((n, D), lambda e, fb: (0, 0)),
        out_shape=jax.ShapeDtypeStruct((n, D), f32),
    )(x, weights, base, fc1_w, fc1_b.reshape(E, 1, FF), fc2_w)

    return (out.reshape(B, NT, D), loss.reshape(()), stats.reshape(E))
